# baseline (speedup 1.0000x reference)
import functools
import math

import jax
import jax.numpy as jnp
from jax import lax
from jax.experimental import pallas as pl
from jax.experimental.pallas import tpu as pltpu
from jax.experimental.pallas import tpu_sc as plsc

H = 256
NCODE = 64
C = 128
NBUF = 2
_INTERPRET = False


def _sc_kernel_body(n, nw, i0, i1, i2, i3, i4, i5, e0, e1, e2, e3, e4, e5, out_hbm,
                    tbl_v, lut_v, dlt_v, idx_v, codes_v, obuf_v,
                    sems_i, sems_o):
    s = math.sqrt(H)
    per_w = n // nw
    nchunk = per_w // C
    wid = lax.axis_index("s") * 2 + lax.axis_index("c")
    base_tok = wid * per_w

    for k, e in enumerate((e0, e1, e2, e3, e4, e5)):
        pltpu.sync_copy(e, tbl_v.at[k])

    def _build(j, carry):
        col = pl.ds(j * 16, 16)
        for k in range(6):
            dlt_v[k, col] = (tbl_v[k, 1, col] - tbl_v[k, 0, col]) * s
        acc = tbl_v[0, 0, col]
        for k in range(1, 6):
            acc = acc + tbl_v[k, 0, col]
        lut_v[0, col] = acc * s
        for k in range(6):
            step = 1 << k
            for c in range(step):
                lut_v[c + step, col] = lut_v[c, col] + dlt_v[k, col]
        return carry

    lax.fori_loop(0, H // 16, _build, 0)

    ihs = (i0, i1, i2, i3, i4, i5)

    def _idx_start(ci, b):
        tok0 = base_tok + ci * C
        for k in range(6):
            pltpu.async_copy(ihs[k].at[pl.ds(tok0, C)], idx_v.at[b, k],
                             sems_i.at[b])

    def _idx_wait(ci, b):
        tok0 = base_tok + ci * C
        for k in range(6):
            pltpu.make_async_copy(ihs[k].at[pl.ds(tok0, C)], idx_v.at[b, k],
                                  sems_i.at[b]).wait()

    for b in range(NBUF):
        _idx_start(b, b)

    def _quad(q, carry):
        ci0 = q * NBUF
        for b in range(NBUF):
            ci = ci0 + b
            tok0 = base_tok + ci * C
            _idx_wait(ci, b)

            @pl.when(q > 0)
            def _wait_prev(_b=b, _ci=ci):
                ptok0 = base_tok + (_ci - NBUF) * C
                pltpu.make_async_copy(obuf_v.at[_b],
                                      out_hbm.at[pl.ds(ptok0, C)],
                                      sems_o.at[_b]).wait()

            @plsc.parallel_loop(0, C // 16, unroll=2)
            def _codes(j, _b=b):
                col = pl.ds(j * 16, 16)
                code = idx_v[_b, 0, col]
                for k in range(1, 6):
                    code = code + idx_v[_b, k, col] * (1 << k)
                codes_v[col] = code

            @plsc.parallel_loop(0, C // 16, unroll=2)
            def _expand(g, _b=b):
                code_vec = codes_v[pl.ds(g * 16, 16)]
                nj = H // 16
                cs = [code_vec[l] for l in range(16)]
                prev = [lut_v[cs[0], pl.ds(j * 16, 16)] for j in range(nj)]
                for l in range(1, 16):
                    cur = []
                    for j in range(nj):
                        cur.append(lut_v[cs[l], pl.ds(j * 16, 16)])
                        obuf_v[_b, g * 16 + l - 1,
                               pl.ds(j * 16, 16)] = prev[j]
                    prev = cur
                for j in range(nj):
                    obuf_v[_b, g * 16 + 15, pl.ds(j * 16, 16)] = prev[j]
            pltpu.async_copy(obuf_v.at[b],
                             out_hbm.at[pl.ds(tok0, C)], sems_o.at[b])
            ci_next = ci + NBUF
            ci_next = jnp.where(ci_next < nchunk, ci_next, ci_next - nchunk)
            _idx_start(ci_next, b)
        return carry

    nquad = nchunk // NBUF
    lax.fori_loop(0, nquad, _quad, 0)

    for b in range(NBUF):
        tok0 = base_tok + ((nquad - 1) * NBUF + b) * C
        pltpu.make_async_copy(obuf_v.at[b],
                              out_hbm.at[pl.ds(tok0, C)],
                              sems_o.at[b]).wait()
        _idx_wait(0, b)


def kernel(mix, falsetto, breathy, pharyngeal, glissando, vibrato,
           mix_emb, falsetto_emb, breathy_emb, pharyngeal_emb,
           glissando_emb, vibrato_emb):
    B, L = mix.shape
    n = B * L
    nw = 32
    idxs = [a.reshape(n) for a in
            (mix, falsetto, breathy, pharyngeal, glissando, vibrato)]
    embs = (mix_emb, falsetto_emb, breathy_emb, pharyngeal_emb,
            glissando_emb, vibrato_emb)
    mesh = plsc.VectorSubcoreMesh(core_axis_name="c", subcore_axis_name="s")
    body = functools.partial(_sc_kernel_body, n, nw)
    out = pl.kernel(
        body,
        out_type=jax.ShapeDtypeStruct((n, H), jnp.float32),
        mesh=mesh,
        scratch_types=[
            pltpu.VMEM((6, 2, H), jnp.float32),
            pltpu.VMEM((NCODE, H), jnp.float32),
            pltpu.VMEM((6, H), jnp.float32),
            pltpu.VMEM((NBUF, 6, C), jnp.int32),
            pltpu.VMEM((C,), jnp.int32),
            pltpu.VMEM((NBUF, C, H), jnp.float32),
            pltpu.SemaphoreType.DMA((NBUF,)),
            pltpu.SemaphoreType.DMA((NBUF,)),
        ],
        interpret=_INTERPRET,
    )(*idxs, *embs)
    return out.reshape(B, L, H)


# speedup vs baseline: 1.0024x; 1.0024x over previous
import functools
import math

import jax
import jax.numpy as jnp
from jax import lax
from jax.experimental import pallas as pl
from jax.experimental.pallas import tpu as pltpu
from jax.experimental.pallas import tpu_sc as plsc

H = 256
NCODE = 64
C = 128
NBUF = 2


def _sc_kernel_body(n, nw, i0, i1, i2, i3, i4, i5, e0, e1, e2, e3, e4, e5, out_hbm,
                    tbl_v, lut_v, dlt_v, idx_v, codes_v, obuf_v,
                    sems_i, sems_o):
    s = math.sqrt(H)
    per_w = n // nw
    nchunk = per_w // C
    wid = lax.axis_index("s") * 2 + lax.axis_index("c")
    base_tok = wid * per_w

    for k, e in enumerate((e0, e1, e2, e3, e4, e5)):
        pltpu.sync_copy(e, tbl_v.at[k])

    def _build(j, carry):
        col = pl.ds(j * 16, 16)
        for k in range(6):
            dlt_v[k, col] = (tbl_v[k, 1, col] - tbl_v[k, 0, col]) * s
        acc = tbl_v[0, 0, col]
        for k in range(1, 6):
            acc = acc + tbl_v[k, 0, col]
        lut_v[0, col] = acc * s
        for k in range(6):
            step = 1 << k
            for c in range(step):
                lut_v[c + step, col] = lut_v[c, col] + dlt_v[k, col]
        return carry

    lax.fori_loop(0, H // 16, _build, 0)

    ihs = (i0, i1, i2, i3, i4, i5)

    def _idx_start(ci, b):
        tok0 = base_tok + ci * C
        for k in range(6):
            pltpu.async_copy(ihs[k].at[pl.ds(tok0, C)], idx_v.at[b, k],
                             sems_i.at[b])

    def _idx_wait(ci, b):
        tok0 = base_tok + ci * C
        for k in range(6):
            pltpu.make_async_copy(ihs[k].at[pl.ds(tok0, C)], idx_v.at[b, k],
                                  sems_i.at[b]).wait()

    for b in range(NBUF):
        _idx_start(b, b)

    def _quad(q, carry):
        ci0 = q * NBUF
        for b in range(NBUF):
            ci = ci0 + b
            tok0 = base_tok + ci * C
            _idx_wait(ci, b)

            @pl.when(q > 0)
            def _wait_prev(_b=b, _ci=ci):
                ptok0 = base_tok + (_ci - NBUF) * C
                pltpu.make_async_copy(obuf_v.at[_b],
                                      out_hbm.at[pl.ds(ptok0, C)],
                                      sems_o.at[_b]).wait()

            @plsc.parallel_loop(0, C // 16, unroll=2)
            def _codes(j, _b=b):
                col = pl.ds(j * 16, 16)
                code = idx_v[_b, 0, col]
                for k in range(1, 6):
                    code = code + idx_v[_b, k, col] * (1 << k)
                codes_v[col] = code

            @plsc.parallel_loop(0, C // 16, unroll=2)
            def _expand(g, _b=b):
                code_vec = codes_v[pl.ds(g * 16, 16)]
                nj = H // 16
                cs = [code_vec[l] for l in range(16)]
                prev = [lut_v[cs[0], pl.ds(j * 16, 16)] for j in range(nj)]
                for l in range(1, 16):
                    cur = []
                    for j in range(nj):
                        cur.append(lut_v[cs[l], pl.ds(j * 16, 16)])
                        obuf_v[_b, g * 16 + l - 1,
                               pl.ds(j * 16, 16)] = prev[j]
                    prev = cur
                for j in range(nj):
                    obuf_v[_b, g * 16 + 15, pl.ds(j * 16, 16)] = prev[j]
            pltpu.async_copy(obuf_v.at[b],
                             out_hbm.at[pl.ds(tok0, C)], sems_o.at[b])
            ci_next = ci + NBUF
            ci_next = jnp.where(ci_next < nchunk, ci_next, ci_next - nchunk)
            _idx_start(ci_next, b)
        return carry

    nquad = nchunk // NBUF
    lax.fori_loop(0, nquad, _quad, 0)

    for b in range(NBUF):
        tok0 = base_tok + ((nquad - 1) * NBUF + b) * C
        pltpu.make_async_copy(obuf_v.at[b],
                              out_hbm.at[pl.ds(tok0, C)],
                              sems_o.at[b]).wait()
        _idx_wait(0, b)


def kernel(mix, falsetto, breathy, pharyngeal, glissando, vibrato,
           mix_emb, falsetto_emb, breathy_emb, pharyngeal_emb,
           glissando_emb, vibrato_emb):
    B, L = mix.shape
    n = B * L
    nw = 32
    idxs = [a.reshape(n) for a in
            (mix, falsetto, breathy, pharyngeal, glissando, vibrato)]
    embs = (mix_emb, falsetto_emb, breathy_emb, pharyngeal_emb,
            glissando_emb, vibrato_emb)
    mesh = plsc.VectorSubcoreMesh(core_axis_name="c", subcore_axis_name="s")
    body = functools.partial(_sc_kernel_body, n, nw)
    out = pl.kernel(
        body,
        out_type=jax.ShapeDtypeStruct((n, H), jnp.float32),
        mesh=mesh,
        scratch_types=[
            pltpu.VMEM((6, 2, H), jnp.float32),
            pltpu.VMEM((NCODE, H), jnp.float32),
            pltpu.VMEM((6, H), jnp.float32),
            pltpu.VMEM((NBUF, 6, C), jnp.int32),
            pltpu.VMEM((C,), jnp.int32),
            pltpu.VMEM((NBUF, C, H), jnp.float32),
            pltpu.SemaphoreType.DMA((NBUF,)),
            pltpu.SemaphoreType.DMA((NBUF,)),
        ],
    )(*idxs, *embs)
    return out.reshape(B, L, H)


# speedup vs baseline: 1.0057x; 1.0034x over previous
import functools
import math

import jax
import jax.numpy as jnp
from jax import lax
from jax.experimental import pallas as pl
from jax.experimental.pallas import tpu as pltpu
from jax.experimental.pallas import tpu_sc as plsc

H = 256
NCODE = 64
C = 128
NBUF = 2


def _sc_kernel_body(n, nw, i0, i1, i2, i3, i4, i5, e0, e1, e2, e3, e4, e5, out_hbm,
                    tbl_v, lut_v, dlt_v, idx_v, codes_v, obuf_v,
                    sems_i, sems_o):
    s = math.sqrt(H)
    per_w = n // nw
    nchunk = per_w // C
    wid = lax.axis_index("s") * 2 + lax.axis_index("c")
    base_tok = wid * per_w

    for k, e in enumerate((e0, e1, e2, e3, e4, e5)):
        pltpu.sync_copy(e, tbl_v.at[k])

    def _build(j, carry):
        col = pl.ds(j * 16, 16)
        for k in range(6):
            dlt_v[k, col] = (tbl_v[k, 1, col] - tbl_v[k, 0, col]) * s
        acc = tbl_v[0, 0, col]
        for k in range(1, 6):
            acc = acc + tbl_v[k, 0, col]
        lut_v[0, col] = acc * s
        for k in range(6):
            step = 1 << k
            for c in range(step):
                lut_v[c + step, col] = lut_v[c, col] + dlt_v[k, col]
        return carry

    lax.fori_loop(0, H // 16, _build, 0)

    ihs = (i0, i1, i2, i3, i4, i5)

    def _idx_start(ci, b):
        tok0 = base_tok + ci * C
        for k in range(6):
            pltpu.async_copy(ihs[k].at[pl.ds(tok0, C)], idx_v.at[b, k],
                             sems_i.at[b])

    def _idx_wait(ci, b):
        tok0 = base_tok + ci * C
        for k in range(6):
            pltpu.make_async_copy(ihs[k].at[pl.ds(tok0, C)], idx_v.at[b, k],
                                  sems_i.at[b]).wait()

    for b in range(NBUF):
        _idx_start(b, b)

    def _quad(q, carry):
        ci0 = q * NBUF
        for b in range(NBUF):
            ci = ci0 + b
            tok0 = base_tok + ci * C
            _idx_wait(ci, b)

            @pl.when(q > 0)
            def _wait_prev(_b=b, _ci=ci):
                ptok0 = base_tok + (_ci - NBUF) * C
                pltpu.make_async_copy(obuf_v.at[_b],
                                      out_hbm.at[pl.ds(ptok0, C)],
                                      sems_o.at[_b]).wait()

            @plsc.parallel_loop(0, C // 16, unroll=2)
            def _expand(g, _b=b):
                col = pl.ds(g * 16, 16)
                code_vec = idx_v[_b, 0, col]
                for k in range(1, 6):
                    code_vec = code_vec + idx_v[_b, k, col] * (1 << k)
                nj = H // 16
                cs = [code_vec[l] for l in range(16)]
                prev = [lut_v[cs[0], pl.ds(j * 16, 16)] for j in range(nj)]
                for l in range(1, 16):
                    cur = []
                    for j in range(nj):
                        cur.append(lut_v[cs[l], pl.ds(j * 16, 16)])
                        obuf_v[_b, g * 16 + l - 1,
                               pl.ds(j * 16, 16)] = prev[j]
                    prev = cur
                for j in range(nj):
                    obuf_v[_b, g * 16 + 15, pl.ds(j * 16, 16)] = prev[j]
            pltpu.async_copy(obuf_v.at[b],
                             out_hbm.at[pl.ds(tok0, C)], sems_o.at[b])
            ci_next = ci + NBUF
            ci_next = jnp.where(ci_next < nchunk, ci_next, ci_next - nchunk)
            _idx_start(ci_next, b)
        return carry

    nquad = nchunk // NBUF
    lax.fori_loop(0, nquad, _quad, 0)

    for b in range(NBUF):
        tok0 = base_tok + ((nquad - 1) * NBUF + b) * C
        pltpu.make_async_copy(obuf_v.at[b],
                              out_hbm.at[pl.ds(tok0, C)],
                              sems_o.at[b]).wait()
        _idx_wait(0, b)


def kernel(mix, falsetto, breathy, pharyngeal, glissando, vibrato,
           mix_emb, falsetto_emb, breathy_emb, pharyngeal_emb,
           glissando_emb, vibrato_emb):
    B, L = mix.shape
    n = B * L
    nw = 32
    idxs = [a.reshape(n) for a in
            (mix, falsetto, breathy, pharyngeal, glissando, vibrato)]
    embs = (mix_emb, falsetto_emb, breathy_emb, pharyngeal_emb,
            glissando_emb, vibrato_emb)
    mesh = plsc.VectorSubcoreMesh(core_axis_name="c", subcore_axis_name="s")
    body = functools.partial(_sc_kernel_body, n, nw)
    out = pl.kernel(
        body,
        out_type=jax.ShapeDtypeStruct((n, H), jnp.float32),
        mesh=mesh,
        scratch_types=[
            pltpu.VMEM((6, 2, H), jnp.float32),
            pltpu.VMEM((NCODE, H), jnp.float32),
            pltpu.VMEM((6, H), jnp.float32),
            pltpu.VMEM((NBUF, 6, C), jnp.int32),
            pltpu.VMEM((C,), jnp.int32),
            pltpu.VMEM((NBUF, C, H), jnp.float32),
            pltpu.SemaphoreType.DMA((NBUF,)),
            pltpu.SemaphoreType.DMA((NBUF,)),
        ],
    )(*idxs, *embs)
    return out.reshape(B, L, H)
